# Initial kernel scaffold; baseline (speedup 1.0000x reference)
import jax
import jax.numpy as jnp
from jax.experimental import pallas as pl


def kernel(x, router_w, w13, w2):
    raise NotImplementedError("write your pallas kernel here")


# speedup vs baseline: 8.0343x; 8.0343x over previous
import functools

import jax
import jax.numpy as jnp
from jax.experimental import pallas as pl

_B, _S, _HIDDEN, _INTER, _E, _K = 8, 4, 512, 1024, 8, 2
_T = _B * _S


def _moe_kernel(x_ref, rw_ref, w13_ref, w2_ref, out_ref):
    e = pl.program_id(0)
    x = x_ref[...]

    logits = jnp.dot(x, rw_ref[...].T, preferred_element_type=jnp.float32)
    probs = jax.nn.softmax(logits, axis=-1)
    i1 = jnp.argmax(probs, axis=-1)
    m1 = jnp.max(probs, axis=-1)
    eidx = jax.lax.broadcasted_iota(jnp.int32, probs.shape, 1)
    masked = jnp.where(eidx == i1[:, None], -jnp.inf, probs)
    i2 = jnp.argmax(masked, axis=-1)
    m2 = jnp.max(masked, axis=-1)
    denom = m1 + m2 + 1e-6
    coef = (jnp.where(i1 == e, m1, 0.0) + jnp.where(i2 == e, m2, 0.0)) / denom

    h = jnp.dot(x, w13_ref[0], preferred_element_type=jnp.float32)
    gate = h[:, :_INTER]
    up = h[:, _INTER:]
    act = gate * jax.nn.sigmoid(gate) * up
    o = jnp.dot(act, w2_ref[0], preferred_element_type=jnp.float32)
    contrib = o * coef[:, None]

    @pl.when(e == 0)
    def _init():
        out_ref[...] = contrib

    @pl.when(e != 0)
    def _acc():
        out_ref[...] += contrib


@jax.jit
def kernel(x, router_w, w13, w2):
    xf = x.reshape(_T, _HIDDEN)
    out = pl.pallas_call(
        _moe_kernel,
        grid=(_E,),
        in_specs=[
            pl.BlockSpec((_T, _HIDDEN), lambda e: (0, 0)),
            pl.BlockSpec((_E, _HIDDEN), lambda e: (0, 0)),
            pl.BlockSpec((1, _HIDDEN, 2 * _INTER), lambda e: (e, 0, 0)),
            pl.BlockSpec((1, _INTER, _HIDDEN), lambda e: (e, 0, 0)),
        ],
        out_specs=pl.BlockSpec((_T, _HIDDEN), lambda e: (0, 0)),
        out_shape=jax.ShapeDtypeStruct((_T, _HIDDEN), jnp.float32),
    )(xf, router_w, w13, w2)
    return out.reshape(_B, _S, _HIDDEN)
